# Initial kernel scaffold; baseline (speedup 1.0000x reference)
import jax
import jax.numpy as jnp
from jax.experimental import pallas as pl


def kernel(weight):
    raise NotImplementedError("write your pallas kernel here")


# speedup vs baseline: 1.6967x; 1.6967x over previous
import functools

import jax
import jax.numpy as jnp
from jax import lax
from jax.experimental import pallas as pl
from jax.experimental.pallas import tpu as pltpu
from jax.experimental.pallas import tpu_sc as plsc

N = 5308416
NTILES = 32
PER_TILE = N // NTILES
CHUNK = 5184
NCHUNKS = PER_TILE // CHUNK
L = 16

K_MAIN = int(N * 0.05)
K_FALL = max(1, int(N * 0.01))

PASSES = ((21, 11), (10, 11), (0, 10))


def _mesh():
    return plsc.VectorSubcoreMesh(core_axis_name="c", subcore_axis_name="s")


def _wid():
    return lax.axis_index("s") * 2 + lax.axis_index("c")


def _dma_start(src, dst, sem):
    pltpu.make_async_copy(src, dst, sem).start()


def _dma_wait(src, dst, sem):
    pltpu.make_async_copy(src, dst, sem).wait()


def _make_hist_kernel(shift, bits, ntargets):
    nb = 1 << bits
    hsz = ntargets * nb * L
    hi_shift = shift + bits

    def body(*refs):
        if ntargets == 1:
            w_hbm, out_hbm, buf0, buf1, hist, red, sem0, sem1 = refs
            state_v = None
        else:
            w_hbm, state_hbm, out_hbm, buf0, buf1, hist, red, sbuf, sem0, sem1 = refs
            state_v = sbuf
        wid = _wid()
        base = wid * PER_TILE

        zeros = jnp.zeros((L,), jnp.int32)
        ones = jnp.ones((L,), jnp.int32)
        iota = jnp.arange(L, dtype=jnp.int32)
        lane_off = iota * nb

        def zero_body(i, _):
            hist[pl.ds(i * L, L)] = zeros
            return 0
        lax.fori_loop(0, hsz // L, zero_body, 0)

        if ntargets == 2:
            pltpu.sync_copy(state_hbm, sbuf)
            p0 = jnp.max(sbuf[0])
            p1 = jnp.max(sbuf[3])
            p0v = jnp.full((L,), p0, jnp.int32)
            p1v = jnp.full((L,), p1, jnp.int32)

        def process(buf):
            def inner(j, _):
                for u in range(4):
                    v = buf[pl.ds(j * (4 * L) + u * L, L)]
                    k = plsc.bitcast(v, jnp.int32) & jnp.int32(0x7FFFFFFF)
                    b = lax.shift_right_logical(k, jnp.int32(shift))
                    if bits < 32 - shift:
                        b = b & jnp.int32(nb - 1)
                    idx = b + lane_off
                    if ntargets == 1:
                        plsc.addupdate_scatter(hist, [idx], ones)
                    else:
                        hi = lax.shift_right_logical(k, jnp.int32(hi_shift))
                        plsc.addupdate_scatter(hist, [idx], ones, mask=hi == p0v)
                        plsc.addupdate_scatter(hist, [idx + jnp.int32(nb * L)],
                                               ones, mask=hi == p1v)
                return 0
            lax.fori_loop(0, CHUNK // (4 * L), inner, 0)

        _dma_start(w_hbm.at[pl.ds(base, CHUNK)], buf0, sem0)
        _dma_start(w_hbm.at[pl.ds(base + CHUNK, CHUNK)], buf1, sem1)

        def outer(g, _):
            off = base + g * (2 * CHUNK)
            _dma_wait(w_hbm.at[pl.ds(off, CHUNK)], buf0, sem0)
            process(buf0)

            @pl.when(g < NCHUNKS // 2 - 1)
            def _():
                _dma_start(w_hbm.at[pl.ds(off + 2 * CHUNK, CHUNK)], buf0, sem0)

            _dma_wait(w_hbm.at[pl.ds(off + CHUNK, CHUNK)], buf1, sem1)
            process(buf1)

            @pl.when(g < NCHUNKS // 2 - 1)
            def _():
                _dma_start(w_hbm.at[pl.ds(off + 3 * CHUNK, CHUNK)], buf1, sem1)
            return 0
        lax.fori_loop(0, NCHUNKS // 2, outer, 0)

        for t in range(ntargets):
            def red_body(c, _, t=t):
                acc = jnp.zeros((L,), jnp.int32)
                for l in range(L):
                    acc = acc + hist[pl.ds(t * nb * L + l * nb + c * L, L)]
                red[pl.ds(t * nb + c * L, L)] = acc
                return 0
            lax.fori_loop(0, nb // L, red_body, 0)

        for t in range(ntargets):
            pltpu.sync_copy(red.at[pl.ds(t * nb, nb)],
                            out_hbm.at[t * NTILES + wid])

    scratch = [
        pltpu.VMEM((CHUNK,), jnp.float32),
        pltpu.VMEM((CHUNK,), jnp.float32),
        pltpu.VMEM((hsz,), jnp.int32),
        pltpu.VMEM((ntargets * nb,), jnp.int32),
    ]
    if ntargets == 2:
        scratch.append(pltpu.VMEM((8, L), jnp.int32))
    scratch += [pltpu.SemaphoreType.DMA, pltpu.SemaphoreType.DMA]

    return pl.kernel(
        body,
        out_type=jax.ShapeDtypeStruct((ntargets * NTILES, nb), jnp.int32),
        mesh=_mesh(),
        scratch_types=scratch,
        compiler_params=pltpu.CompilerParams(needs_layout_passes=False),
        name=f"sc_hist_s{shift}_b{bits}_t{ntargets}",
    )


def _make_walk_kernel(bits, shared_hist, final):
    nb = 1 << bits

    def body(hists_hbm, state_hbm, out_state_hbm, *rest):
        if final:
            out_final_hbm = rest[0]
            rest = rest[1:]
        tbuf, ghist, sbuf, obuf = rest[:4]
        fbuf = rest[4] if final else None

        @pl.when(_wid() == 0)
        def _():
            pltpu.sync_copy(state_hbm, sbuf)
            new_scalars = []
            for t in range(2):
                prefix = jnp.max(sbuf[3 * t + 0])
                rank = jnp.max(sbuf[3 * t + 1])
                bacc = jnp.max(sbuf[3 * t + 2])

                rows0 = 0 if shared_hist else t * NTILES
                pltpu.sync_copy(hists_hbm.at[pl.ds(rows0, NTILES)], tbuf)

                def sum_body(c, _):
                    acc = jnp.zeros((L,), jnp.int32)
                    for r in range(NTILES):
                        acc = acc + tbuf[r, pl.ds(c * L, L)]
                    ghist[pl.ds(c * L, L)] = acc
                    return 0
                lax.fori_loop(0, nb // L, sum_body, 0)

                def walk_body(c, carry):
                    tot, binv, belowv = carry
                    h = ghist[pl.ds(c * L, L)]
                    cum = plsc.cumsum(h) + tot
                    m = cum <= rank
                    binv = binv + jnp.where(m, jnp.int32(1), jnp.int32(0))
                    belowv = belowv + jnp.where(m, h, jnp.int32(0))
                    return (tot + jnp.sum(h), binv, belowv)

                z = jnp.zeros((L,), jnp.int32)
                _, binv, belowv = lax.fori_loop(
                    0, nb // L, walk_body, (jnp.int32(0), z, z))
                binidx = jnp.sum(binv)
                below = jnp.sum(belowv)

                prefix = lax.shift_left(prefix, jnp.int32(bits)) | binidx
                rank = rank - below
                bacc = bacc + below
                new_scalars += [prefix, rank, bacc]

            for i, s in enumerate(new_scalars):
                obuf[i] = jnp.full((L,), s, jnp.int32)
            obuf[6] = jnp.zeros((L,), jnp.int32)
            obuf[7] = jnp.zeros((L,), jnp.int32)
            pltpu.sync_copy(obuf, out_state_hbm)

            if final:
                key_fall, _, _, key_main, _, below_main = new_scalars
                use_fall = below_main == jnp.int32(0)
                thr_key = jnp.where(use_fall, key_fall, key_main)
                count = jnp.where(use_fall, jnp.int32(K_FALL), below_main)
                thr = plsc.bitcast(jnp.full((L,), thr_key, jnp.int32),
                                   jnp.float32)
                count_v = jnp.full((L,), count, jnp.int32).astype(jnp.float32)
                ratio_v = jnp.clip(count_v / jnp.full((L,), N, jnp.float32),
                                   0.01, 0.05)
                fbuf[0] = thr
                fbuf[1] = ratio_v
                for i in range(2, 8):
                    fbuf[i] = jnp.zeros((L,), jnp.float32)
                pltpu.sync_copy(fbuf, out_final_hbm)

    out_type = [jax.ShapeDtypeStruct((8, L), jnp.int32)]
    if final:
        out_type.append(jax.ShapeDtypeStruct((8, L), jnp.float32))
    scratch = [
        pltpu.VMEM((NTILES, nb), jnp.int32),
        pltpu.VMEM((nb,), jnp.int32),
        pltpu.VMEM((8, L), jnp.int32),
        pltpu.VMEM((8, L), jnp.int32),
    ]
    if final:
        scratch.append(pltpu.VMEM((8, L), jnp.float32))

    return pl.kernel(
        body,
        out_type=tuple(out_type) if final else out_type[0],
        mesh=_mesh(),
        scratch_types=scratch,
        compiler_params=pltpu.CompilerParams(needs_layout_passes=False),
        name=f"sc_walk_b{bits}{'_final' if final else ''}",
    )


def _mask_body(thr_ref, w_ref, o_ref):
    thr = thr_ref[0, 0]
    w = w_ref[...]
    o_ref[...] = w * (jnp.abs(w) >= thr).astype(jnp.float32)


def _mask_tc(w2d, thr):
    rows, cols = w2d.shape
    block = rows // 8
    return pl.pallas_call(
        _mask_body,
        grid=(8,),
        in_specs=[
            pl.BlockSpec(memory_space=pltpu.SMEM),
            pl.BlockSpec((block, cols), lambda i: (i, 0)),
        ],
        out_specs=pl.BlockSpec((block, cols), lambda i: (i, 0)),
        out_shape=jax.ShapeDtypeStruct((rows, cols), jnp.float32),
    )(thr, w2d)


def kernel(weight):
    wf = weight.reshape(-1)
    w2d = weight.reshape(5184, 1024)

    state0 = jnp.tile(
        jnp.array([0, K_FALL, 0, 0, K_MAIN, 0, 0, 0],
                  jnp.int32)[:, None], (1, L))

    hist1 = _make_hist_kernel(*PASSES[0], ntargets=1)
    hist2 = _make_hist_kernel(*PASSES[1], ntargets=2)
    hist3 = _make_hist_kernel(*PASSES[2], ntargets=2)
    walk1 = _make_walk_kernel(PASSES[0][1], shared_hist=True, final=False)
    walk2 = _make_walk_kernel(PASSES[1][1], shared_hist=False, final=False)
    walk3 = _make_walk_kernel(PASSES[2][1], shared_hist=False, final=True)

    h1 = hist1(wf)
    s1 = walk1(h1, state0)
    h2 = hist2(wf, s1)
    s2 = walk2(h2, s1)
    h3 = hist3(wf, s2)
    _, fin = walk3(h3, s2)

    thr = fin[0:1, 0:1]
    masked = _mask_tc(w2d, thr)
    return masked.reshape(768, 768, 3, 3), fin[1, 0]


# speedup vs baseline: 35.0130x; 20.6358x over previous
import functools

import jax
import jax.numpy as jnp
from jax import lax
from jax.experimental import pallas as pl
from jax.experimental.pallas import tpu as pltpu
from jax.experimental.pallas import tpu_sc as plsc

N = 5308416
NTILES = 32
PER_TILE = N // NTILES
CHUNK = 5184
NCHUNKS = PER_TILE // CHUNK
L = 16

K_MAIN = int(N * 0.05)
K_FALL = max(1, int(N * 0.01))

PASSES = ((21, 11), (10, 11), (0, 10))


def _mesh():
    return plsc.VectorSubcoreMesh(core_axis_name="c", subcore_axis_name="s")


def _wid():
    return lax.axis_index("s") * 2 + lax.axis_index("c")


def _dma_start(src, dst, sem):
    pltpu.make_async_copy(src, dst, sem).start()


def _dma_wait(src, dst, sem):
    pltpu.make_async_copy(src, dst, sem).wait()


def _make_hist_kernel(shift, bits, ntargets):
    nb = 1 << bits
    hsz = ntargets * nb * L
    hi_shift = shift + bits

    def body(*refs):
        if ntargets == 1:
            w_hbm, out_hbm, buf0, buf1, hist, red, sem0, sem1 = refs
            state_v = None
        else:
            w_hbm, state_hbm, out_hbm, buf0, buf1, hist, red, sbuf, sem0, sem1 = refs
            state_v = sbuf
        wid = _wid()
        base = wid * PER_TILE

        zeros = jnp.zeros((L,), jnp.int32)
        ones = jnp.ones((L,), jnp.int32)
        iota = jnp.arange(L, dtype=jnp.int32)
        lane_off = iota * nb

        def zero_body(i, _):
            hist[pl.ds(i * L, L)] = zeros
            return 0
        lax.fori_loop(0, hsz // L, zero_body, 0)

        if ntargets == 2:
            pltpu.sync_copy(state_hbm, sbuf)
            p0 = jnp.max(sbuf[0])
            p1 = jnp.max(sbuf[3])
            p0v = jnp.full((L,), p0, jnp.int32)
            p1v = jnp.full((L,), p1, jnp.int32)

        def process(buf):
            def inner(j, _):
                for u in range(4):
                    v = buf[pl.ds(j * (4 * L) + u * L, L)]
                    k = plsc.bitcast(v, jnp.int32) & jnp.int32(0x7FFFFFFF)
                    b = lax.shift_right_logical(k, jnp.int32(shift))
                    if bits < 32 - shift:
                        b = b & jnp.int32(nb - 1)
                    idx = b + lane_off
                    if ntargets == 1:
                        plsc.addupdate_scatter(hist, [idx], ones)
                    else:
                        hi = lax.shift_right_logical(k, jnp.int32(hi_shift))
                        plsc.addupdate_scatter(hist, [idx], ones, mask=hi == p0v)
                        plsc.addupdate_scatter(hist, [idx + jnp.int32(nb * L)],
                                               ones, mask=hi == p1v)
                return 0
            lax.fori_loop(0, CHUNK // (4 * L), inner, 0)

        _dma_start(w_hbm.at[pl.ds(base, CHUNK)], buf0, sem0)
        _dma_start(w_hbm.at[pl.ds(base + CHUNK, CHUNK)], buf1, sem1)

        def outer(g, _):
            off = base + g * (2 * CHUNK)
            _dma_wait(w_hbm.at[pl.ds(off, CHUNK)], buf0, sem0)
            process(buf0)

            @pl.when(g < NCHUNKS // 2 - 1)
            def _():
                _dma_start(w_hbm.at[pl.ds(off + 2 * CHUNK, CHUNK)], buf0, sem0)

            _dma_wait(w_hbm.at[pl.ds(off + CHUNK, CHUNK)], buf1, sem1)
            process(buf1)

            @pl.when(g < NCHUNKS // 2 - 1)
            def _():
                _dma_start(w_hbm.at[pl.ds(off + 3 * CHUNK, CHUNK)], buf1, sem1)
            return 0
        lax.fori_loop(0, NCHUNKS // 2, outer, 0)

        for t in range(ntargets):
            def red_body(c, _, t=t):
                acc = jnp.zeros((L,), jnp.int32)
                for l in range(L):
                    acc = acc + hist[pl.ds(t * nb * L + l * nb + c * L, L)]
                red[pl.ds(t * nb + c * L, L)] = acc
                return 0
            lax.fori_loop(0, nb // L, red_body, 0)

        for t in range(ntargets):
            pltpu.sync_copy(red.at[pl.ds(t * nb, nb)],
                            out_hbm.at[t * NTILES + wid])

    scratch = [
        pltpu.VMEM((CHUNK,), jnp.float32),
        pltpu.VMEM((CHUNK,), jnp.float32),
        pltpu.VMEM((hsz,), jnp.int32),
        pltpu.VMEM((ntargets * nb,), jnp.int32),
    ]
    if ntargets == 2:
        scratch.append(pltpu.VMEM((8, L), jnp.int32))
    scratch += [pltpu.SemaphoreType.DMA, pltpu.SemaphoreType.DMA]

    return pl.kernel(
        body,
        out_type=jax.ShapeDtypeStruct((ntargets * NTILES, nb), jnp.int32),
        mesh=_mesh(),
        scratch_types=scratch,
        compiler_params=pltpu.CompilerParams(needs_layout_passes=False),
        name=f"sc_hist_s{shift}_b{bits}_t{ntargets}",
    )


def _make_walk_kernel(bits, shared_hist, final):
    nb = 1 << bits

    def body(hists_hbm, state_hbm, out_state_hbm, *rest):
        if final:
            out_final_hbm = rest[0]
            rest = rest[1:]
        tbuf, ghist, sbuf, obuf = rest[:4]
        fbuf = rest[4] if final else None

        @pl.when(_wid() == 0)
        def _():
            pltpu.sync_copy(state_hbm, sbuf)
            new_scalars = []
            for t in range(2):
                prefix = jnp.max(sbuf[3 * t + 0])
                rank = jnp.max(sbuf[3 * t + 1])
                bacc = jnp.max(sbuf[3 * t + 2])

                rows0 = 0 if shared_hist else t * NTILES
                pltpu.sync_copy(hists_hbm.at[pl.ds(rows0, NTILES)], tbuf)

                def sum_body(c, _):
                    acc = jnp.zeros((L,), jnp.int32)
                    for r in range(NTILES):
                        acc = acc + tbuf[r, pl.ds(c * L, L)]
                    ghist[pl.ds(c * L, L)] = acc
                    return 0
                lax.fori_loop(0, nb // L, sum_body, 0)

                def walk_body(c, carry):
                    tot, binv, belowv = carry
                    h = ghist[pl.ds(c * L, L)]
                    cum = plsc.cumsum(h) + tot
                    m = cum <= rank
                    binv = binv + jnp.where(m, jnp.int32(1), jnp.int32(0))
                    belowv = belowv + jnp.where(m, h, jnp.int32(0))
                    return (tot + jnp.sum(h), binv, belowv)

                z = jnp.zeros((L,), jnp.int32)
                _, binv, belowv = lax.fori_loop(
                    0, nb // L, walk_body, (jnp.int32(0), z, z))
                binidx = jnp.sum(binv)
                below = jnp.sum(belowv)

                prefix = lax.shift_left(prefix, jnp.int32(bits)) | binidx
                rank = rank - below
                bacc = bacc + below
                new_scalars += [prefix, rank, bacc]

            for i, s in enumerate(new_scalars):
                obuf[i] = jnp.full((L,), s, jnp.int32)
            obuf[6] = jnp.zeros((L,), jnp.int32)
            obuf[7] = jnp.zeros((L,), jnp.int32)
            pltpu.sync_copy(obuf, out_state_hbm)

            if final:
                key_fall, _, _, key_main, _, below_main = new_scalars
                use_fall = below_main == jnp.int32(0)
                thr_key = jnp.where(use_fall, key_fall, key_main)
                count = jnp.where(use_fall, jnp.int32(K_FALL), below_main)
                thr = plsc.bitcast(jnp.full((L,), thr_key, jnp.int32),
                                   jnp.float32)
                count_v = jnp.full((L,), count, jnp.int32).astype(jnp.float32)
                ratio_v = jnp.clip(count_v / jnp.full((L,), N, jnp.float32),
                                   0.01, 0.05)
                fbuf[0] = thr
                fbuf[1] = ratio_v
                for i in range(2, 8):
                    fbuf[i] = jnp.zeros((L,), jnp.float32)
                pltpu.sync_copy(fbuf, out_final_hbm)

    out_type = [jax.ShapeDtypeStruct((8, L), jnp.int32)]
    if final:
        out_type.append(jax.ShapeDtypeStruct((8, L), jnp.float32))
    scratch = [
        pltpu.VMEM((NTILES, nb), jnp.int32),
        pltpu.VMEM((nb,), jnp.int32),
        pltpu.VMEM((8, L), jnp.int32),
        pltpu.VMEM((8, L), jnp.int32),
    ]
    if final:
        scratch.append(pltpu.VMEM((8, L), jnp.float32))

    return pl.kernel(
        body,
        out_type=tuple(out_type) if final else out_type[0],
        mesh=_mesh(),
        scratch_types=scratch,
        compiler_params=pltpu.CompilerParams(needs_layout_passes=False),
        name=f"sc_walk_b{bits}{'_final' if final else ''}",
    )


def _mask_body(thr_ref, w_ref, o_ref):
    thr = thr_ref[0, 0]
    w = w_ref[...]
    o_ref[...] = w * (jnp.abs(w) >= thr).astype(jnp.float32)


def _mask_tc(w2d, thr):
    rows, cols = w2d.shape
    block = rows // 8
    return pl.pallas_call(
        _mask_body,
        grid=(8,),
        in_specs=[
            pl.BlockSpec(memory_space=pltpu.SMEM),
            pl.BlockSpec((block, cols), lambda i: (i, 0)),
        ],
        out_specs=pl.BlockSpec((block, cols), lambda i: (i, 0)),
        out_shape=jax.ShapeDtypeStruct((rows, cols), jnp.float32),
    )(thr, w2d)


def kernel(weight):
    wp = jnp.transpose(weight, (2, 3, 0, 1)).reshape(6912, 768)
    wf = wp.reshape(-1)

    state0 = jnp.tile(
        jnp.array([0, K_FALL, 0, 0, K_MAIN, 0, 0, 0],
                  jnp.int32)[:, None], (1, L))

    hist1 = _make_hist_kernel(*PASSES[0], ntargets=1)
    hist2 = _make_hist_kernel(*PASSES[1], ntargets=2)
    hist3 = _make_hist_kernel(*PASSES[2], ntargets=2)
    walk1 = _make_walk_kernel(PASSES[0][1], shared_hist=True, final=False)
    walk2 = _make_walk_kernel(PASSES[1][1], shared_hist=False, final=False)
    walk3 = _make_walk_kernel(PASSES[2][1], shared_hist=False, final=True)

    h1 = hist1(wf)
    s1 = walk1(h1, state0)
    h2 = hist2(wf, s1)
    s2 = walk2(h2, s1)
    h3 = hist3(wf, s2)
    _, fin = walk3(h3, s2)

    thr = fin[0:1, 0:1]
    masked_p = _mask_tc(wp, thr)
    masked = jnp.transpose(masked_p.reshape(3, 3, 768, 768), (2, 3, 0, 1))
    return masked, fin[1, 0]


# speedup vs baseline: 74.3840x; 2.1245x over previous
import functools

import jax
import jax.numpy as jnp
from jax import lax
from jax.experimental import pallas as pl
from jax.experimental.pallas import tpu as pltpu
from jax.experimental.pallas import tpu_sc as plsc

N = 5308416
NTILES = 32
PER_TILE = N // NTILES
CHUNK = 5184
NCHUNKS = PER_TILE // CHUNK
L = 16

K_MAIN = int(N * 0.05)
K_FALL = max(1, int(N * 0.01))

PASSES = ((21, 11), (10, 11), (0, 10))


def _mesh():
    return plsc.VectorSubcoreMesh(core_axis_name="c", subcore_axis_name="s")


def _wid():
    return lax.axis_index("s") * 2 + lax.axis_index("c")


def _dma_start(src, dst, sem):
    pltpu.make_async_copy(src, dst, sem).start()


def _dma_wait(src, dst, sem):
    pltpu.make_async_copy(src, dst, sem).wait()


def _make_hist_kernel(shift, bits, ntargets):
    nb = 1 << bits
    hsz = ntargets * nb * L
    hi_shift = shift + bits

    def body(*refs):
        if ntargets == 1:
            w_hbm, out_hbm, buf0, buf1, hist, red, sem0, sem1 = refs
            state_v = None
        else:
            w_hbm, state_hbm, out_hbm, buf0, buf1, hist, red, sbuf, sem0, sem1 = refs
            state_v = sbuf
        wid = _wid()
        base = wid * PER_TILE

        zeros = jnp.zeros((L,), jnp.int32)
        ones = jnp.ones((L,), jnp.int32)
        iota = jnp.arange(L, dtype=jnp.int32)
        lane_off = iota * nb

        @plsc.parallel_loop(0, hsz // L, unroll=8)
        def _(i):
            hist[pl.ds(i * L, L)] = zeros

        if ntargets == 2:
            pltpu.sync_copy(state_hbm, sbuf)
            p0 = jnp.max(sbuf[0])
            p1 = jnp.max(sbuf[3])
            p0v = jnp.full((L,), p0, jnp.int32)
            p1v = jnp.full((L,), p1, jnp.int32)

        def process(buf):
            @plsc.parallel_loop(0, CHUNK // L, unroll=8)
            def _(j):
                v = buf[pl.ds(j * L, L)]
                k = plsc.bitcast(v, jnp.int32) & jnp.int32(0x7FFFFFFF)
                b = lax.shift_right_logical(k, jnp.int32(shift))
                if bits < 32 - shift:
                    b = b & jnp.int32(nb - 1)
                idx = b + lane_off
                if ntargets == 1:
                    plsc.addupdate_scatter(hist, [idx], ones)
                else:
                    hi = lax.shift_right_logical(k, jnp.int32(hi_shift))
                    plsc.addupdate_scatter(hist, [idx], ones, mask=hi == p0v)
                    plsc.addupdate_scatter(hist, [idx + jnp.int32(nb * L)],
                                           ones, mask=hi == p1v)

        _dma_start(w_hbm.at[pl.ds(base, CHUNK)], buf0, sem0)
        _dma_start(w_hbm.at[pl.ds(base + CHUNK, CHUNK)], buf1, sem1)

        def outer(g, _):
            off = base + g * (2 * CHUNK)
            _dma_wait(w_hbm.at[pl.ds(off, CHUNK)], buf0, sem0)
            process(buf0)

            @pl.when(g < NCHUNKS // 2 - 1)
            def _():
                _dma_start(w_hbm.at[pl.ds(off + 2 * CHUNK, CHUNK)], buf0, sem0)

            _dma_wait(w_hbm.at[pl.ds(off + CHUNK, CHUNK)], buf1, sem1)
            process(buf1)

            @pl.when(g < NCHUNKS // 2 - 1)
            def _():
                _dma_start(w_hbm.at[pl.ds(off + 3 * CHUNK, CHUNK)], buf1, sem1)
            return 0
        lax.fori_loop(0, NCHUNKS // 2, outer, 0)

        for t in range(ntargets):
            @plsc.parallel_loop(0, nb // L, unroll=2)
            def _(c, t=t):
                acc = jnp.zeros((L,), jnp.int32)
                for l in range(L):
                    acc = acc + hist[pl.ds(t * nb * L + l * nb + c * L, L)]
                red[pl.ds(t * nb + c * L, L)] = acc

        for t in range(ntargets):
            pltpu.sync_copy(red.at[pl.ds(t * nb, nb)],
                            out_hbm.at[t * NTILES + wid])

    scratch = [
        pltpu.VMEM((CHUNK,), jnp.float32),
        pltpu.VMEM((CHUNK,), jnp.float32),
        pltpu.VMEM((hsz,), jnp.int32),
        pltpu.VMEM((ntargets * nb,), jnp.int32),
    ]
    if ntargets == 2:
        scratch.append(pltpu.VMEM((8, L), jnp.int32))
    scratch += [pltpu.SemaphoreType.DMA, pltpu.SemaphoreType.DMA]

    return pl.kernel(
        body,
        out_type=jax.ShapeDtypeStruct((ntargets * NTILES, nb), jnp.int32),
        mesh=_mesh(),
        scratch_types=scratch,
        compiler_params=pltpu.CompilerParams(needs_layout_passes=False),
        name=f"sc_hist_s{shift}_b{bits}_t{ntargets}",
    )


def _make_walk_kernel(bits, shared_hist, final):
    nb = 1 << bits

    def body(hists_hbm, state_hbm, out_state_hbm, *rest):
        if final:
            out_final_hbm = rest[0]
            rest = rest[1:]
        tbuf, ghist, sbuf, obuf = rest[:4]
        fbuf = rest[4] if final else None

        @pl.when(_wid() == 0)
        def _():
            pltpu.sync_copy(state_hbm, sbuf)
            new_scalars = []
            for t in range(2):
                prefix = jnp.max(sbuf[3 * t + 0])
                rank = jnp.max(sbuf[3 * t + 1])
                bacc = jnp.max(sbuf[3 * t + 2])

                rows0 = 0 if shared_hist else t * NTILES
                pltpu.sync_copy(hists_hbm.at[pl.ds(rows0, NTILES)], tbuf)

                def sum_body(c, _):
                    acc = jnp.zeros((L,), jnp.int32)
                    for r in range(NTILES):
                        acc = acc + tbuf[r, pl.ds(c * L, L)]
                    ghist[pl.ds(c * L, L)] = acc
                    return 0
                lax.fori_loop(0, nb // L, sum_body, 0)

                def walk_body(c, carry):
                    tot, binv, belowv = carry
                    h = ghist[pl.ds(c * L, L)]
                    cum = plsc.cumsum(h) + tot
                    m = cum <= rank
                    binv = binv + jnp.where(m, jnp.int32(1), jnp.int32(0))
                    belowv = belowv + jnp.where(m, h, jnp.int32(0))
                    return (tot + jnp.sum(h), binv, belowv)

                z = jnp.zeros((L,), jnp.int32)
                _, binv, belowv = lax.fori_loop(
                    0, nb // L, walk_body, (jnp.int32(0), z, z))
                binidx = jnp.sum(binv)
                below = jnp.sum(belowv)

                prefix = lax.shift_left(prefix, jnp.int32(bits)) | binidx
                rank = rank - below
                bacc = bacc + below
                new_scalars += [prefix, rank, bacc]

            for i, s in enumerate(new_scalars):
                obuf[i] = jnp.full((L,), s, jnp.int32)
            obuf[6] = jnp.zeros((L,), jnp.int32)
            obuf[7] = jnp.zeros((L,), jnp.int32)
            pltpu.sync_copy(obuf, out_state_hbm)

            if final:
                key_fall, _, _, key_main, _, below_main = new_scalars
                use_fall = below_main == jnp.int32(0)
                thr_key = jnp.where(use_fall, key_fall, key_main)
                count = jnp.where(use_fall, jnp.int32(K_FALL), below_main)
                thr = plsc.bitcast(jnp.full((L,), thr_key, jnp.int32),
                                   jnp.float32)
                count_v = jnp.full((L,), count, jnp.int32).astype(jnp.float32)
                ratio_v = jnp.clip(count_v / jnp.full((L,), N, jnp.float32),
                                   0.01, 0.05)
                fbuf[0] = thr
                fbuf[1] = ratio_v
                for i in range(2, 8):
                    fbuf[i] = jnp.zeros((L,), jnp.float32)
                pltpu.sync_copy(fbuf, out_final_hbm)

    out_type = [jax.ShapeDtypeStruct((8, L), jnp.int32)]
    if final:
        out_type.append(jax.ShapeDtypeStruct((8, L), jnp.float32))
    scratch = [
        pltpu.VMEM((NTILES, nb), jnp.int32),
        pltpu.VMEM((nb,), jnp.int32),
        pltpu.VMEM((8, L), jnp.int32),
        pltpu.VMEM((8, L), jnp.int32),
    ]
    if final:
        scratch.append(pltpu.VMEM((8, L), jnp.float32))

    return pl.kernel(
        body,
        out_type=tuple(out_type) if final else out_type[0],
        mesh=_mesh(),
        scratch_types=scratch,
        compiler_params=pltpu.CompilerParams(needs_layout_passes=False),
        name=f"sc_walk_b{bits}{'_final' if final else ''}",
    )


def _mask_body(thr_ref, w_ref, o_ref):
    thr = thr_ref[0, 0]
    w = w_ref[...]
    o_ref[...] = w * (jnp.abs(w) >= thr).astype(jnp.float32)


def _mask_tc(w2d, thr):
    rows, cols = w2d.shape
    block = rows // 8
    return pl.pallas_call(
        _mask_body,
        grid=(8,),
        in_specs=[
            pl.BlockSpec(memory_space=pltpu.SMEM),
            pl.BlockSpec((block, cols), lambda i: (i, 0)),
        ],
        out_specs=pl.BlockSpec((block, cols), lambda i: (i, 0)),
        out_shape=jax.ShapeDtypeStruct((rows, cols), jnp.float32),
    )(thr, w2d)


def kernel(weight):
    wp = jnp.transpose(weight, (2, 3, 0, 1)).reshape(6912, 768)
    wf = wp.reshape(-1)

    state0 = jnp.tile(
        jnp.array([0, K_FALL, 0, 0, K_MAIN, 0, 0, 0],
                  jnp.int32)[:, None], (1, L))

    hist1 = _make_hist_kernel(*PASSES[0], ntargets=1)
    hist2 = _make_hist_kernel(*PASSES[1], ntargets=2)
    hist3 = _make_hist_kernel(*PASSES[2], ntargets=2)
    walk1 = _make_walk_kernel(PASSES[0][1], shared_hist=True, final=False)
    walk2 = _make_walk_kernel(PASSES[1][1], shared_hist=False, final=False)
    walk3 = _make_walk_kernel(PASSES[2][1], shared_hist=False, final=True)

    h1 = hist1(wf)
    s1 = walk1(h1, state0)
    h2 = hist2(wf, s1)
    s2 = walk2(h2, s1)
    h3 = hist3(wf, s2)
    _, fin = walk3(h3, s2)

    thr = fin[0:1, 0:1]
    masked_p = _mask_tc(wp, thr)
    masked = jnp.transpose(masked_p.reshape(3, 3, 768, 768), (2, 3, 0, 1))
    return masked, fin[1, 0]
